# baseline (speedup 1.0000x reference)
import jax
import jax.numpy as jnp
from jax.experimental import pallas as pl
from jax.experimental.pallas import tpu as pltpu

_N_REAL = 1723
_N_PAD = 1728
_EPS = 1e-5


def _enc_kernel(x_ref, w_ref, b_ref, o_ref):
    @pl.when(pl.program_id(1) == 0)
    def _init():
        o_ref[...] = jnp.broadcast_to(b_ref[...], o_ref.shape)

    o_ref[...] += jax.lax.dot_general(
        x_ref[0], w_ref[0], (((1,), (1,)), ((), ())),
        preferred_element_type=jnp.float32)


def _base_kernel(e_ref, w_ref, b_ref, o_ref):
    o_ref[...] = jnp.dot(e_ref[...], w_ref[...],
                         preferred_element_type=jnp.float32) + b_ref[...]


def _gn_relu(x, gamma, beta):
    n, c = x.shape
    g = c // 8
    cnt = 8.0 * _N_REAL
    npad = float(_N_PAD - _N_REAL)
    rows = jax.lax.broadcasted_iota(jnp.int32, (c, g), 0) // 8
    cols = jax.lax.broadcasted_iota(jnp.int32, (c, g), 1)
    P = (rows == cols).astype(jnp.float32)
    rows_t = jax.lax.broadcasted_iota(jnp.int32, (g, c), 0)
    cols_t = jax.lax.broadcasted_iota(jnp.int32, (g, c), 1) // 8
    PT = (rows_t == cols_t).astype(jnp.float32)

    hi = jax.lax.Precision.HIGHEST
    pv = x[_N_REAL:_N_REAL + 1, :]
    s = jnp.sum(x, axis=0, keepdims=True) - npad * pv
    q = jnp.sum(x * x, axis=0, keepdims=True) - npad * (pv * pv)
    mean_g = jnp.dot(s, P, precision=hi, preferred_element_type=jnp.float32) / cnt
    ex2_g = jnp.dot(q, P, precision=hi, preferred_element_type=jnp.float32) / cnt
    var_g = ex2_g - mean_g * mean_g
    inv_g = jax.lax.rsqrt(var_g + _EPS)
    scale_c = jnp.dot(inv_g, PT, precision=hi,
                      preferred_element_type=jnp.float32) * gamma
    shift_c = beta - jnp.dot(mean_g * inv_g, PT, precision=hi,
                             preferred_element_type=jnp.float32) * gamma
    return jnp.maximum(x * scale_c + shift_c, 0.0).astype(jnp.bfloat16)


def _mm(a, b):
    return jnp.dot(a, b, preferred_element_type=jnp.float32)


def _make_trunk(meta):
    def body(*refs):
        out_ref = refs[-1]
        it = iter(refs[:-1])
        base_ref = next(it)
        rv_ref = next(it)
        a_ref = next(it)
        wrv_ref = next(it)

        A = a_ref[...]

        h = _mm(rv_ref[...], wrv_ref[...]) + base_ref[0]

        for has_skip in meta:
            pre_g = next(it)[...]
            pre_b = next(it)[...]
            lin1_wt = next(it)[...]
            lin1_b = next(it)[...]
            n1_g = next(it)[...]
            n1_b = next(it)[...]
            conv_w = next(it)[...]
            conv_b = next(it)[...]
            n2_g = next(it)[...]
            n2_b = next(it)[...]
            lin2_wt = next(it)[...]
            lin2_b = next(it)[...]
            y = _gn_relu(h, pre_g, pre_b)
            y = _mm(y, lin1_wt) + lin1_b
            y = _gn_relu(y, n1_g, n1_b)
            y = _mm(A, _mm(y, conv_w).astype(jnp.bfloat16)) + conv_b
            y = _gn_relu(y, n2_g, n2_b)
            y = _mm(y, lin2_wt) + lin2_b
            if has_skip:
                skip_wt = next(it)[...]
                skip_b = next(it)[...]
                h = _mm(h.astype(jnp.bfloat16), skip_wt) + skip_b
            h = h + y

        fin_g = next(it)[...]
        fin_b = next(it)[...]
        out_wt = next(it)[...]
        out_b = next(it)[...]
        y = _gn_relu(h, fin_g, fin_b)
        out_ref[0] = _mm(y, out_wt) + out_b

    return body


def _row(v):
    return v.reshape(1, -1)


def kernel(x, params, A, ref_vertices):
    f32 = jnp.float32
    B = x.shape[0]
    n = A.shape[0]
    pad_n = _N_PAD - n

    w4 = params['inconv_W']
    o_dim, c_dim = w4.shape[0], w4.shape[1]
    wt = jnp.transpose(w4, (2, 3, 0, 1)).reshape(49, o_dim, c_dim)
    x4 = jnp.pad(x, ((0, 8 - B), (0, 0), (0, 0), (0, 0)))
    xt = jnp.transpose(x4, (2, 3, 0, 1)).reshape(49, 8, c_dim)
    o_blk = 2048
    enc = pl.pallas_call(
        _enc_kernel,
        grid=(o_dim // o_blk, 49),
        in_specs=[
            pl.BlockSpec((1, 8, c_dim), lambda o, p: (p, 0, 0)),
            pl.BlockSpec((1, o_blk, c_dim), lambda o, p: (p, o, 0)),
            pl.BlockSpec((1, o_blk), lambda o, p: (0, o)),
        ],
        out_specs=pl.BlockSpec((8, o_blk), lambda o, p: (0, o)),
        out_shape=jax.ShapeDtypeStruct((8, o_dim), f32),
    )(xt, wt, _row(params['inconv_b']))

    w_enc_t = params['gl0_W'][:, 3:].T
    base = pl.pallas_call(
        _base_kernel,
        out_shape=jax.ShapeDtypeStruct((8, w_enc_t.shape[1]), f32),
    )(enc, w_enc_t, _row(params['gl0_b']))[:B].reshape(B, 1, -1)

    bf16 = jnp.bfloat16
    rv = jnp.pad(ref_vertices.T, ((0, pad_n), (0, 5))).astype(bf16)
    w_rv_t = jnp.pad(params['gl0_W'][:, :3].T, ((0, 5), (0, 0))).astype(bf16)
    a_pad = jnp.pad(A, ((0, pad_n), (0, pad_n))).astype(jnp.bfloat16)

    wlist, meta = [], []
    for p in params['gc'] + params['shape']:
        has_skip = 'skip_W' in p
        meta.append(has_skip)
        wlist += [
            _row(p['pre_g']), _row(p['pre_b']),
            p['lin1_W'].T.astype(bf16), _row(p['lin1_b']),
            _row(p['n1_g']), _row(p['n1_b']),
            p['conv_W'].astype(bf16), _row(p['conv_b']),
            _row(p['n2_g']), _row(p['n2_b']),
            p['lin2_W'].T.astype(bf16), _row(p['lin2_b']),
        ]
        if has_skip:
            wlist += [p['skip_W'].T.astype(bf16), _row(p['skip_b'])]
    out_wt = jnp.pad(params['out_W'].T, ((0, 0), (0, 5))).astype(bf16)
    out_b = jnp.pad(_row(params['out_b']), ((0, 0), (0, 5)))
    wlist += [_row(params['final_g']), _row(params['final_b']), out_wt, out_b]

    const = lambda b: (0, 0)
    in_specs = [
        pl.BlockSpec((1, 1, base.shape[2]), lambda b: (b, 0, 0)),
        pl.BlockSpec(rv.shape, const),
        pl.BlockSpec(a_pad.shape, const),
        pl.BlockSpec(w_rv_t.shape, const),
    ] + [pl.BlockSpec(w.shape, const) for w in wlist]

    out = pl.pallas_call(
        _make_trunk(meta),
        grid=(B,),
        in_specs=in_specs,
        out_specs=pl.BlockSpec((1, _N_PAD, 8), lambda b: (b, 0, 0)),
        out_shape=jax.ShapeDtypeStruct((B, _N_PAD, 8), f32),
        compiler_params=pltpu.CompilerParams(
            vmem_limit_bytes=100 * 1024 * 1024),
    )(base, rv, a_pad, w_rv_t, *wlist)

    return out[:, :n, :3]


# speedup vs baseline: 1.0125x; 1.0125x over previous
import jax
import jax.numpy as jnp
from jax.experimental import pallas as pl
from jax.experimental.pallas import tpu as pltpu

_N_REAL = 1723
_N_PAD = 1728
_EPS = 1e-5


def _enc_kernel(x_ref, w_ref, b_ref, o_ref):
    @pl.when(pl.program_id(1) == 0)
    def _init():
        o_ref[...] = jnp.broadcast_to(b_ref[...], o_ref.shape)

    o_ref[...] += jax.lax.dot_general(
        x_ref[0], w_ref[0], (((1,), (1,)), ((), ())),
        preferred_element_type=jnp.float32)


def _base_kernel(e_ref, w_ref, b_ref, o_ref):
    o_ref[...] = jnp.dot(e_ref[...], w_ref[...],
                         preferred_element_type=jnp.float32) + b_ref[...]


def _gn_relu(x, gamma, beta):
    n, c = x.shape
    g = c // 8
    cnt = 8.0 * _N_REAL
    npad = float(_N_PAD - _N_REAL)
    rows = jax.lax.broadcasted_iota(jnp.int32, (c, g), 0) // 8
    cols = jax.lax.broadcasted_iota(jnp.int32, (c, g), 1)
    P = (rows == cols).astype(jnp.float32)
    rows_t = jax.lax.broadcasted_iota(jnp.int32, (g, c), 0)
    cols_t = jax.lax.broadcasted_iota(jnp.int32, (g, c), 1) // 8
    PT = (rows_t == cols_t).astype(jnp.float32)

    hi = jax.lax.Precision.HIGHEST
    pv = x[_N_REAL:_N_REAL + 1, :]
    s = jnp.sum(x, axis=0, keepdims=True) - npad * pv
    q = jnp.sum(x * x, axis=0, keepdims=True) - npad * (pv * pv)
    mean_g = jnp.dot(s, P, precision=hi, preferred_element_type=jnp.float32) / cnt
    ex2_g = jnp.dot(q, P, precision=hi, preferred_element_type=jnp.float32) / cnt
    var_g = ex2_g - mean_g * mean_g
    inv_g = jax.lax.rsqrt(var_g + _EPS)
    scale_c = jnp.dot(inv_g, PT, precision=hi,
                      preferred_element_type=jnp.float32) * gamma
    shift_c = beta - jnp.dot(mean_g * inv_g, PT, precision=hi,
                             preferred_element_type=jnp.float32) * gamma
    return jnp.maximum(x * scale_c + shift_c, 0.0)


def _mm(a, b):
    return jnp.dot(a, b, preferred_element_type=jnp.float32)


def _make_trunk(meta):
    def body(*refs):
        out_ref = refs[-1]
        it = iter(refs[:-1])
        base_ref = next(it)
        rv_ref = next(it)
        a_ref = next(it)
        wrv_ref = next(it)

        A = a_ref[...]

        h = _mm(rv_ref[...], wrv_ref[...]) + base_ref[0]

        for has_skip in meta:
            pre_g = next(it)[...]
            pre_b = next(it)[...]
            lin1_wt = next(it)[...]
            lin1_b = next(it)[...]
            n1_g = next(it)[...]
            n1_b = next(it)[...]
            conv_w = next(it)[...]
            conv_b = next(it)[...]
            n2_g = next(it)[...]
            n2_b = next(it)[...]
            lin2_wt = next(it)[...]
            lin2_b = next(it)[...]
            y = _gn_relu(h, pre_g, pre_b)
            y = _mm(y, lin1_wt) + lin1_b
            y = _gn_relu(y, n1_g, n1_b)
            y = _mm(A, _mm(y, conv_w).astype(jnp.bfloat16)) + conv_b
            y = _gn_relu(y, n2_g, n2_b)
            y = _mm(y, lin2_wt) + lin2_b
            if has_skip:
                skip_wt = next(it)[...]
                skip_b = next(it)[...]
                h = _mm(h, skip_wt) + skip_b
            h = h + y

        fin_g = next(it)[...]
        fin_b = next(it)[...]
        out_wt = next(it)[...]
        out_b = next(it)[...]
        y = _gn_relu(h, fin_g, fin_b)
        out_ref[0] = _mm(y, out_wt) + out_b

    return body


def _row(v):
    return v.reshape(1, -1)


def kernel(x, params, A, ref_vertices):
    f32 = jnp.float32
    B = x.shape[0]
    n = A.shape[0]
    pad_n = _N_PAD - n

    w4 = params['inconv_W']
    o_dim, c_dim = w4.shape[0], w4.shape[1]
    wt = jnp.transpose(w4, (2, 3, 0, 1)).reshape(49, o_dim, c_dim)
    x4 = jnp.pad(x, ((0, 8 - B), (0, 0), (0, 0), (0, 0)))
    xt = jnp.transpose(x4, (2, 3, 0, 1)).reshape(49, 8, c_dim)
    o_blk = 2048
    enc = pl.pallas_call(
        _enc_kernel,
        grid=(o_dim // o_blk, 49),
        in_specs=[
            pl.BlockSpec((1, 8, c_dim), lambda o, p: (p, 0, 0)),
            pl.BlockSpec((1, o_blk, c_dim), lambda o, p: (p, o, 0)),
            pl.BlockSpec((1, o_blk), lambda o, p: (0, o)),
        ],
        out_specs=pl.BlockSpec((8, o_blk), lambda o, p: (0, o)),
        out_shape=jax.ShapeDtypeStruct((8, o_dim), f32),
    )(xt, wt, _row(params['inconv_b']))

    w_enc_t = params['gl0_W'][:, 3:].T
    base = pl.pallas_call(
        _base_kernel,
        out_shape=jax.ShapeDtypeStruct((8, w_enc_t.shape[1]), f32),
    )(enc, w_enc_t, _row(params['gl0_b']))[:B].reshape(B, 1, -1)

    bf16 = jnp.bfloat16
    rv = jnp.pad(ref_vertices.T, ((0, pad_n), (0, 5))).astype(bf16)
    w_rv_t = jnp.pad(params['gl0_W'][:, :3].T, ((0, 5), (0, 0))).astype(bf16)
    a_pad = jnp.pad(A, ((0, pad_n), (0, pad_n))).astype(jnp.bfloat16)

    wlist, meta = [], []
    for p in params['gc'] + params['shape']:
        has_skip = 'skip_W' in p
        meta.append(has_skip)
        wlist += [
            _row(p['pre_g']), _row(p['pre_b']),
            p['lin1_W'].T.astype(bf16), _row(p['lin1_b']),
            _row(p['n1_g']), _row(p['n1_b']),
            p['conv_W'].astype(bf16), _row(p['conv_b']),
            _row(p['n2_g']), _row(p['n2_b']),
            p['lin2_W'].T.astype(bf16), _row(p['lin2_b']),
        ]
        if has_skip:
            wlist += [p['skip_W'].T.astype(bf16), _row(p['skip_b'])]
    out_wt = jnp.pad(params['out_W'].T, ((0, 0), (0, 5))).astype(bf16)
    out_b = jnp.pad(_row(params['out_b']), ((0, 0), (0, 5)))
    wlist += [_row(params['final_g']), _row(params['final_b']), out_wt, out_b]

    const = lambda b: (0, 0)
    in_specs = [
        pl.BlockSpec((1, 1, base.shape[2]), lambda b: (b, 0, 0)),
        pl.BlockSpec(rv.shape, const),
        pl.BlockSpec(a_pad.shape, const),
        pl.BlockSpec(w_rv_t.shape, const),
    ] + [pl.BlockSpec(w.shape, const) for w in wlist]

    out = pl.pallas_call(
        _make_trunk(meta),
        grid=(B,),
        in_specs=in_specs,
        out_specs=pl.BlockSpec((1, _N_PAD, 8), lambda b: (b, 0, 0)),
        out_shape=jax.ShapeDtypeStruct((B, _N_PAD, 8), f32),
        compiler_params=pltpu.CompilerParams(
            vmem_limit_bytes=100 * 1024 * 1024),
    )(base, rv, a_pad, w_rv_t, *wlist)

    return out[:, :n, :3]


# speedup vs baseline: 1.2061x; 1.1912x over previous
import jax
import jax.numpy as jnp
from jax.experimental import pallas as pl
from jax.experimental.pallas import tpu as pltpu

_N_REAL = 1723
_N_PAD = 1728
_EPS = 1e-5


def _enc_kernel(x_ref, w_ref, b_ref, o_ref):
    @pl.when(pl.program_id(1) == 0)
    def _init():
        o_ref[...] = jnp.broadcast_to(b_ref[...], o_ref.shape)

    o_ref[...] += jax.lax.dot_general(
        x_ref[0], w_ref[0], (((1,), (1,)), ((), ())),
        preferred_element_type=jnp.float32)


def _base_kernel(e_ref, w_ref, b_ref, o_ref):
    o_ref[...] = jnp.dot(e_ref[...], w_ref[...],
                         preferred_element_type=jnp.float32) + b_ref[...]


def _gn_relu(x, gamma, beta):
    n, c = x.shape
    g = c // 8
    cnt = 8.0 * _N_REAL
    npad = float(_N_PAD - _N_REAL)
    rows = jax.lax.broadcasted_iota(jnp.int32, (c, g), 0) // 8
    cols = jax.lax.broadcasted_iota(jnp.int32, (c, g), 1)
    P = (rows == cols).astype(jnp.float32)
    rows_t = jax.lax.broadcasted_iota(jnp.int32, (g, c), 0)
    cols_t = jax.lax.broadcasted_iota(jnp.int32, (g, c), 1) // 8
    PT = (rows_t == cols_t).astype(jnp.float32)

    hi = jax.lax.Precision.HIGHEST
    pv = x[_N_REAL:_N_REAL + 1, :]
    s = jnp.sum(x, axis=0, keepdims=True) - npad * pv
    q = jnp.sum(x * x, axis=0, keepdims=True) - npad * (pv * pv)
    mean_g = jnp.dot(s, P, precision=hi, preferred_element_type=jnp.float32) / cnt
    ex2_g = jnp.dot(q, P, precision=hi, preferred_element_type=jnp.float32) / cnt
    var_g = ex2_g - mean_g * mean_g
    inv_g = jax.lax.rsqrt(var_g + _EPS)
    scale_c = jnp.dot(inv_g, PT, precision=hi,
                      preferred_element_type=jnp.float32) * gamma
    shift_c = beta - jnp.dot(mean_g * inv_g, PT, precision=hi,
                             preferred_element_type=jnp.float32) * gamma
    return jnp.maximum(x * scale_c + shift_c, 0.0)


def _mm(a, b):
    return jnp.dot(a, b, preferred_element_type=jnp.float32)


def _make_trunk(meta, pair):
    def body(*refs):
        out_ref = refs[-1]
        it = iter(refs[:-1])
        base_ref = next(it)
        rv_ref = next(it)
        a_ref = next(it)
        wrv_ref = next(it)

        A = a_ref[...]
        base2 = base_ref[0]
        rvw = _mm(rv_ref[...], wrv_ref[...])

        hs = [rvw + base2[b:b + 1, :] for b in range(pair)]

        for has_skip in meta:
            pre_g = next(it)[...]
            pre_b = next(it)[...]
            lin1_wt = next(it)[...]
            lin1_b = next(it)[...]
            n1_g = next(it)[...]
            n1_b = next(it)[...]
            conv_w = next(it)[...]
            conv_b = next(it)[...]
            n2_g = next(it)[...]
            n2_b = next(it)[...]
            lin2_wt = next(it)[...]
            lin2_b = next(it)[...]
            mid = conv_w.shape[1]
            zs = []
            for b in range(pair):
                y = _gn_relu(hs[b], pre_g, pre_b)
                y = _mm(y, lin1_wt) + lin1_b
                y = _gn_relu(y, n1_g, n1_b)
                zs.append(_mm(y, conv_w).astype(jnp.bfloat16))
            az = _mm(A, jnp.concatenate(zs, axis=1))
            if has_skip:
                skip_wt = next(it)[...]
                skip_b = next(it)[...]
            for b in range(pair):
                y = az[:, b * mid:(b + 1) * mid] + conv_b
                y = _gn_relu(y, n2_g, n2_b)
                y = _mm(y, lin2_wt) + lin2_b
                if has_skip:
                    hs[b] = _mm(hs[b], skip_wt) + skip_b
                hs[b] = hs[b] + y

        fin_g = next(it)[...]
        fin_b = next(it)[...]
        out_wt = next(it)[...]
        out_b = next(it)[...]
        for b in range(pair):
            y = _gn_relu(hs[b], fin_g, fin_b)
            out_ref[b] = _mm(y, out_wt) + out_b

    return body


def _row(v):
    return v.reshape(1, -1)


def kernel(x, params, A, ref_vertices):
    f32 = jnp.float32
    B = x.shape[0]
    n = A.shape[0]
    pad_n = _N_PAD - n

    w4 = params['inconv_W']
    o_dim, c_dim = w4.shape[0], w4.shape[1]
    wt = jnp.transpose(w4, (2, 3, 0, 1)).reshape(49, o_dim, c_dim)
    x4 = jnp.pad(x, ((0, 8 - B), (0, 0), (0, 0), (0, 0)))
    xt = jnp.transpose(x4, (2, 3, 0, 1)).reshape(49, 8, c_dim)
    o_blk = 2048
    enc = pl.pallas_call(
        _enc_kernel,
        grid=(o_dim // o_blk, 49),
        in_specs=[
            pl.BlockSpec((1, 8, c_dim), lambda o, p: (p, 0, 0)),
            pl.BlockSpec((1, o_blk, c_dim), lambda o, p: (p, o, 0)),
            pl.BlockSpec((1, o_blk), lambda o, p: (0, o)),
        ],
        out_specs=pl.BlockSpec((8, o_blk), lambda o, p: (0, o)),
        out_shape=jax.ShapeDtypeStruct((8, o_dim), f32),
    )(xt, wt, _row(params['inconv_b']))

    w_enc_t = params['gl0_W'][:, 3:].T
    base = pl.pallas_call(
        _base_kernel,
        out_shape=jax.ShapeDtypeStruct((8, w_enc_t.shape[1]), f32),
    )(enc, w_enc_t, _row(params['gl0_b']))[:B].reshape(B, 1, -1)

    rv = jnp.pad(ref_vertices.T, ((0, pad_n), (0, 5)))
    w_rv_t = jnp.pad(params['gl0_W'][:, :3].T, ((0, 5), (0, 0)))
    a_pad = jnp.pad(A, ((0, pad_n), (0, pad_n))).astype(jnp.bfloat16)

    wlist, meta = [], []
    for p in params['gc'] + params['shape']:
        has_skip = 'skip_W' in p
        meta.append(has_skip)
        wlist += [
            _row(p['pre_g']), _row(p['pre_b']),
            p['lin1_W'].T, _row(p['lin1_b']),
            _row(p['n1_g']), _row(p['n1_b']),
            p['conv_W'], _row(p['conv_b']),
            _row(p['n2_g']), _row(p['n2_b']),
            p['lin2_W'].T, _row(p['lin2_b']),
        ]
        if has_skip:
            wlist += [p['skip_W'].T, _row(p['skip_b'])]
    out_wt = jnp.pad(params['out_W'].T, ((0, 0), (0, 5)))
    out_b = jnp.pad(_row(params['out_b']), ((0, 0), (0, 5)))
    wlist += [_row(params['final_g']), _row(params['final_b']), out_wt, out_b]

    pair = 2
    base = base.reshape(B // pair, pair, -1)
    const = lambda b: (0, 0)
    in_specs = [
        pl.BlockSpec((1, pair, base.shape[2]), lambda b: (b, 0, 0)),
        pl.BlockSpec(rv.shape, const),
        pl.BlockSpec(a_pad.shape, const),
        pl.BlockSpec(w_rv_t.shape, const),
    ] + [pl.BlockSpec(w.shape, const) for w in wlist]

    out = pl.pallas_call(
        _make_trunk(meta, pair),
        grid=(B // pair,),
        in_specs=in_specs,
        out_specs=pl.BlockSpec((pair, _N_PAD, 8), lambda b: (b, 0, 0)),
        out_shape=jax.ShapeDtypeStruct((B, _N_PAD, 8), f32),
        compiler_params=pltpu.CompilerParams(
            vmem_limit_bytes=100 * 1024 * 1024),
    )(base, rv, a_pad, w_rv_t, *wlist)

    return out[:, :n, :3]


# speedup vs baseline: 1.3256x; 1.0991x over previous
import jax
import jax.numpy as jnp
from jax.experimental import pallas as pl
from jax.experimental.pallas import tpu as pltpu

_N_REAL = 1723
_N_PAD = 1728
_EPS = 1e-5


def _enc_kernel(x_ref, w_ref, b_ref, o_ref):
    @pl.when(pl.program_id(1) == 0)
    def _init():
        o_ref[...] = jnp.broadcast_to(b_ref[...], o_ref.shape)

    o_ref[...] += jax.lax.dot_general(
        x_ref[0], w_ref[0], (((1,), (1,)), ((), ())),
        preferred_element_type=jnp.float32)


def _base_kernel(e_ref, w_ref, b_ref, o_ref):
    o_ref[...] = jnp.dot(e_ref[...], w_ref[...],
                         preferred_element_type=jnp.float32) + b_ref[...]


def _gn_relu(x, gamma, beta):
    n, c = x.shape
    g = c // 8
    cnt = 8.0 * _N_REAL
    npad = float(_N_PAD - _N_REAL)
    rows = jax.lax.broadcasted_iota(jnp.int32, (c, g), 0) // 8
    cols = jax.lax.broadcasted_iota(jnp.int32, (c, g), 1)
    P = (rows == cols).astype(jnp.float32)
    rows_t = jax.lax.broadcasted_iota(jnp.int32, (g, c), 0)
    cols_t = jax.lax.broadcasted_iota(jnp.int32, (g, c), 1) // 8
    PT = (rows_t == cols_t).astype(jnp.float32)

    hi = jax.lax.Precision.HIGHEST
    pv = x[_N_REAL:_N_REAL + 1, :]
    s = jnp.sum(x, axis=0, keepdims=True) - npad * pv
    q = jnp.sum(x * x, axis=0, keepdims=True) - npad * (pv * pv)
    mean_g = jnp.dot(s, P, precision=hi, preferred_element_type=jnp.float32) / cnt
    ex2_g = jnp.dot(q, P, precision=hi, preferred_element_type=jnp.float32) / cnt
    var_g = ex2_g - mean_g * mean_g
    inv_g = jax.lax.rsqrt(var_g + _EPS)
    scale_c = jnp.dot(inv_g, PT, precision=hi,
                      preferred_element_type=jnp.float32) * gamma
    shift_c = beta - jnp.dot(mean_g * inv_g, PT, precision=hi,
                             preferred_element_type=jnp.float32) * gamma
    return jnp.maximum(x * scale_c + shift_c, 0.0)


def _mm(a, b):
    return jnp.dot(a, b, preferred_element_type=jnp.float32)


def _make_trunk(meta, pair):
    def body(*refs):
        out_ref = refs[-1]
        it = iter(refs[:-1])
        base_ref = next(it)
        rv_ref = next(it)
        a_ref = next(it)
        wrv_ref = next(it)

        A = a_ref[...]
        base2 = base_ref[0]
        rvw = _mm(rv_ref[...], wrv_ref[...])

        hs = [rvw + base2[b:b + 1, :] for b in range(pair)]

        for has_skip in meta:
            pre_g = next(it)[...]
            pre_b = next(it)[...]
            lin1_wt = next(it)[...]
            lin1_b = next(it)[...]
            n1_g = next(it)[...]
            n1_b = next(it)[...]
            conv_w = next(it)[...]
            conv_b = next(it)[...]
            n2_g = next(it)[...]
            n2_b = next(it)[...]
            lin2_wt = next(it)[...]
            lin2_b = next(it)[...]
            mid = conv_w.shape[1]
            zs = []
            for b in range(pair):
                y = _gn_relu(hs[b], pre_g, pre_b)
                y = _mm(y, lin1_wt) + lin1_b
                y = _gn_relu(y, n1_g, n1_b)
                zs.append(_mm(y, conv_w).astype(jnp.bfloat16))
            az = _mm(A, jnp.concatenate(zs, axis=1))
            if has_skip:
                skip_wt = next(it)[...]
                skip_b = next(it)[...]
            for b in range(pair):
                y = az[:, b * mid:(b + 1) * mid] + conv_b
                y = _gn_relu(y, n2_g, n2_b)
                y = _mm(y, lin2_wt) + lin2_b
                if has_skip:
                    hs[b] = _mm(hs[b], skip_wt) + skip_b
                hs[b] = hs[b] + y

        fin_g = next(it)[...]
        fin_b = next(it)[...]
        out_wt = next(it)[...]
        out_b = next(it)[...]
        for b in range(pair):
            y = _gn_relu(hs[b], fin_g, fin_b)
            out_ref[b] = _mm(y, out_wt) + out_b

    return body


def _row(v):
    return v.reshape(1, -1)


def kernel(x, params, A, ref_vertices):
    f32 = jnp.float32
    B = x.shape[0]
    n = A.shape[0]
    pad_n = _N_PAD - n

    w4 = params['inconv_W']
    o_dim, c_dim = w4.shape[0], w4.shape[1]
    wt = jnp.transpose(w4, (2, 3, 0, 1)).reshape(49, o_dim, c_dim)
    x4 = jnp.pad(x, ((0, 8 - B), (0, 0), (0, 0), (0, 0)))
    xt = jnp.transpose(x4, (2, 3, 0, 1)).reshape(49, 8, c_dim)
    o_blk = 2048
    enc = pl.pallas_call(
        _enc_kernel,
        grid=(o_dim // o_blk, 49),
        in_specs=[
            pl.BlockSpec((1, 8, c_dim), lambda o, p: (p, 0, 0)),
            pl.BlockSpec((1, o_blk, c_dim), lambda o, p: (p, o, 0)),
            pl.BlockSpec((1, o_blk), lambda o, p: (0, o)),
        ],
        out_specs=pl.BlockSpec((8, o_blk), lambda o, p: (0, o)),
        out_shape=jax.ShapeDtypeStruct((8, o_dim), f32),
    )(xt, wt, _row(params['inconv_b']))

    w_enc_t = params['gl0_W'][:, 3:].T
    base = pl.pallas_call(
        _base_kernel,
        out_shape=jax.ShapeDtypeStruct((8, w_enc_t.shape[1]), f32),
    )(enc, w_enc_t, _row(params['gl0_b']))[:B].reshape(B, 1, -1)

    rv = jnp.pad(ref_vertices.T, ((0, pad_n), (0, 5)))
    w_rv_t = jnp.pad(params['gl0_W'][:, :3].T, ((0, 5), (0, 0)))
    a_pad = jnp.pad(A, ((0, pad_n), (0, pad_n))).astype(jnp.bfloat16)

    wlist, meta = [], []
    for p in params['gc'] + params['shape']:
        has_skip = 'skip_W' in p
        meta.append(has_skip)
        wlist += [
            _row(p['pre_g']), _row(p['pre_b']),
            p['lin1_W'].T, _row(p['lin1_b']),
            _row(p['n1_g']), _row(p['n1_b']),
            p['conv_W'], _row(p['conv_b']),
            _row(p['n2_g']), _row(p['n2_b']),
            p['lin2_W'].T, _row(p['lin2_b']),
        ]
        if has_skip:
            wlist += [p['skip_W'].T, _row(p['skip_b'])]
    out_wt = jnp.pad(params['out_W'].T, ((0, 0), (0, 5)))
    out_b = jnp.pad(_row(params['out_b']), ((0, 0), (0, 5)))
    wlist += [_row(params['final_g']), _row(params['final_b']), out_wt, out_b]

    pair = 4
    base = base.reshape(B // pair, pair, -1)
    const = lambda b: (0, 0)
    in_specs = [
        pl.BlockSpec((1, pair, base.shape[2]), lambda b: (b, 0, 0)),
        pl.BlockSpec(rv.shape, const),
        pl.BlockSpec(a_pad.shape, const),
        pl.BlockSpec(w_rv_t.shape, const),
    ] + [pl.BlockSpec(w.shape, const) for w in wlist]

    out = pl.pallas_call(
        _make_trunk(meta, pair),
        grid=(B // pair,),
        in_specs=in_specs,
        out_specs=pl.BlockSpec((pair, _N_PAD, 8), lambda b: (b, 0, 0)),
        out_shape=jax.ShapeDtypeStruct((B, _N_PAD, 8), f32),
        compiler_params=pltpu.CompilerParams(
            vmem_limit_bytes=100 * 1024 * 1024),
    )(base, rv, a_pad, w_rv_t, *wlist)

    return out[:, :n, :3]
